# Initial kernel scaffold; baseline (speedup 1.0000x reference)
import jax
import jax.numpy as jnp
from jax.experimental import pallas as pl


def kernel(x):
    raise NotImplementedError("write your pallas kernel here")


# speedup vs baseline: 39.9518x; 39.9518x over previous
import jax
import jax.numpy as jnp
from jax import lax
from jax.experimental import pallas as pl
from jax.experimental.pallas import tpu as pltpu
from jax.experimental.pallas import tpu_sc as plsc

L = 16
NC, NS = 2, 16
NW = NC * NS

B, C, H, W = 8, 384, 224, 224
N = H * W
ROWS = B * C
ROWS_PER_W = ROWS // NW
K = 5
UNROLL = 8


def _sc_body(x_hbm, out_hbm, buf, res_v):
    wid = lax.axis_index("s") * NC + lax.axis_index("c")
    base_row = wid * ROWS_PER_W
    neg_inf = jnp.full((L,), -jnp.inf, dtype=jnp.float32)
    lanes = lax.iota(jnp.int32, L)

    def row_body(r, res_vec):
        pltpu.sync_copy(x_hbm.at[base_row + r], buf)

        def chunk_body(i, st):
            m1, m2, m3, m4, m5 = st
            for j in range(UNROLL):
                v = buf[pl.ds(i * (UNROLL * L) + j * L, L)]
                t = jnp.maximum(m1, v); v = jnp.minimum(m1, v); m1 = t
                t = jnp.maximum(m2, v); v = jnp.minimum(m2, v); m2 = t
                t = jnp.maximum(m3, v); v = jnp.minimum(m3, v); m3 = t
                t = jnp.maximum(m4, v); v = jnp.minimum(m4, v); m4 = t
                m5 = jnp.maximum(m5, v)
            return (m1, m2, m3, m4, m5)

        m1, m2, m3, m4, m5 = lax.fori_loop(
            0, N // (UNROLL * L), chunk_body, (neg_inf,) * 5)

        dnums = lax.GatherDimensionNumbers(
            offset_dims=(), collapsed_slice_dims=(0,), start_index_map=(0,))

        def bcast_reduce(v, op):
            for s in (1, 2, 4, 8):
                w = lax.gather(
                    v, (lanes ^ s)[:, None], dnums, slice_sizes=(1,),
                    mode=lax.GatherScatterMode.PROMISE_IN_BOUNDS)
                v = op(v, w)
            return v

        acc = jnp.zeros((L,), jnp.float32)
        for _ in range(K):
            gv = bcast_reduce(m1, jnp.maximum)
            cand = jnp.where(m1 == gv, lanes, L)
            sel = bcast_reduce(cand, jnp.minimum) == lanes
            acc = acc + jnp.where(sel, m1, 0.0)
            m1 = jnp.where(sel, m2, m1)
            m2 = jnp.where(sel, m3, m2)
            m3 = jnp.where(sel, m4, m3)
            m4 = jnp.where(sel, m5, m4)
            m5 = jnp.where(sel, neg_inf, m5)
        mean_v = bcast_reduce(acc, jnp.add) / 5.0
        res_vec = jnp.where(lanes == r % L, mean_v, res_vec)

        @pl.when(r % L == L - 1)
        def _store():
            res_v[pl.ds((r // L) * L, L)] = res_vec

        return res_vec

    lax.fori_loop(0, ROWS_PER_W, row_body, jnp.zeros((L,), jnp.float32))
    pltpu.sync_copy(res_v, out_hbm.at[pl.ds(base_row, ROWS_PER_W)])


def kernel(x):
    xf = x.reshape(ROWS, N)
    out = pl.kernel(
        _sc_body,
        out_type=jax.ShapeDtypeStruct((ROWS,), jnp.float32),
        mesh=plsc.VectorSubcoreMesh(core_axis_name="c", subcore_axis_name="s"),
        scratch_types=[
            pltpu.VMEM((N,), jnp.float32),
            pltpu.VMEM((ROWS_PER_W,), jnp.float32),
        ],
    )(xf)
    return out.reshape(B, C)


# speedup vs baseline: 45.8412x; 1.1474x over previous
import jax
import jax.numpy as jnp
from jax import lax
from jax.experimental import pallas as pl
from jax.experimental.pallas import tpu as pltpu
from jax.experimental.pallas import tpu_sc as plsc

L = 16
NC, NS = 2, 16
NW = NC * NS

B, C, H, W = 8, 384, 224, 224
N = H * W
ROWS = B * C
ROWS_PER_W = ROWS // NW
K = 5
UNROLL = 8
NSTATE = 4


def _insert(m, v):
    for k in range(K - 1):
        t = jnp.maximum(m[k], v)
        v = jnp.minimum(m[k], v)
        m[k] = t
    m[K - 1] = jnp.maximum(m[K - 1], v)


def _sc_body(x_hbm, out_hbm, buf0, buf1, res_v, sem0, sem1):
    wid = lax.axis_index("s") * NC + lax.axis_index("c")
    base_row = wid * ROWS_PER_W
    neg_inf = jnp.full((L,), -jnp.inf, dtype=jnp.float32)
    lanes = lax.iota(jnp.int32, L)
    dnums = lax.GatherDimensionNumbers(
        offset_dims=(), collapsed_slice_dims=(0,), start_index_map=(0,))

    def bcast_reduce(v, op):
        for s in (1, 2, 4, 8):
            w = lax.gather(
                v, (lanes ^ s)[:, None], dnums, slice_sizes=(1,),
                mode=lax.GatherScatterMode.PROMISE_IN_BOUNDS)
            v = op(v, w)
        return v

    def process_row(r, buf, res_vec):
        def chunk_body(i, st):
            ms = [list(st[K * s:K * (s + 1)]) for s in range(NSTATE)]
            for j in range(UNROLL):
                v = buf[pl.ds(i * (UNROLL * L) + j * L, L)]
                _insert(ms[j % NSTATE], v)
            return tuple(x for m in ms for x in m)

        st = lax.fori_loop(
            0, N // (UNROLL * L), chunk_body, (neg_inf,) * (K * NSTATE))

        m = list(st[0:K])
        for s in range(1, NSTATE):
            for k in range(K):
                _insert(m, st[K * s + k])
        m1, m2, m3, m4, m5 = m

        acc = jnp.zeros((L,), jnp.float32)
        for _ in range(K):
            gv = bcast_reduce(m1, jnp.maximum)
            cand = jnp.where(m1 == gv, lanes, L)
            sel = bcast_reduce(cand, jnp.minimum) == lanes
            acc = acc + jnp.where(sel, m1, 0.0)
            m1 = jnp.where(sel, m2, m1)
            m2 = jnp.where(sel, m3, m2)
            m3 = jnp.where(sel, m4, m3)
            m4 = jnp.where(sel, m5, m4)
            m5 = jnp.where(sel, neg_inf, m5)
        mean_v = bcast_reduce(acc, jnp.add) / 5.0
        res_vec = jnp.where(lanes == r % L, mean_v, res_vec)

        @pl.when(r % L == L - 1)
        def _store():
            res_v[pl.ds((r // L) * L, L)] = res_vec

        return res_vec

    pltpu.async_copy(x_hbm.at[base_row], buf0, sem0)
    pltpu.async_copy(x_hbm.at[base_row + 1], buf1, sem1)

    def pair_body(g, res_vec):
        r0 = 2 * g
        pltpu.make_async_copy(x_hbm.at[base_row], buf0, sem0).wait()
        res_vec = process_row(r0, buf0, res_vec)

        @pl.when(g < ROWS_PER_W // 2 - 1)
        def _next0():
            pltpu.async_copy(x_hbm.at[base_row + r0 + 2], buf0, sem0)

        pltpu.make_async_copy(x_hbm.at[base_row], buf1, sem1).wait()
        res_vec = process_row(r0 + 1, buf1, res_vec)

        @pl.when(g < ROWS_PER_W // 2 - 1)
        def _next1():
            pltpu.async_copy(x_hbm.at[base_row + r0 + 3], buf1, sem1)

        return res_vec

    lax.fori_loop(0, ROWS_PER_W // 2, pair_body, jnp.zeros((L,), jnp.float32))
    pltpu.sync_copy(res_v, out_hbm.at[pl.ds(base_row, ROWS_PER_W)])


def kernel(x):
    xf = x.reshape(ROWS, N)
    out = pl.kernel(
        _sc_body,
        out_type=jax.ShapeDtypeStruct((ROWS,), jnp.float32),
        mesh=plsc.VectorSubcoreMesh(core_axis_name="c", subcore_axis_name="s"),
        scratch_types=[
            pltpu.VMEM((N,), jnp.float32),
            pltpu.VMEM((N,), jnp.float32),
            pltpu.VMEM((ROWS_PER_W,), jnp.float32),
            pltpu.SemaphoreType.DMA,
            pltpu.SemaphoreType.DMA,
        ],
    )(xf)
    return out.reshape(B, C)


# speedup vs baseline: 133.2414x; 2.9066x over previous
import jax
import jax.numpy as jnp
from jax import lax
from jax.experimental import pallas as pl
from jax.experimental.pallas import tpu as pltpu
from jax.experimental.pallas import tpu_sc as plsc

L = 16
NC, NS = 2, 16

B, C, H, W = 8, 384, 224, 224
CT, CL = C // 128, 128
WT, WS = W // 8, 8
NG = CL // L
NPAIR = B * CT
TASKS_PER_TILE = 3
HQ = H // 4
K = 5
CH = 2
NCHUNK = HQ // CH


def _insert(m, v):
    for k in range(K - 1):
        t = jnp.maximum(m[k], v)
        v = jnp.minimum(m[k], v)
        m[k] = t
    m[K - 1] = jnp.maximum(m[K - 1], v)


def _scan_body(x_hbm, cands_hbm, buf0, buf1, cand_v, sem0, sem1):
    core = lax.axis_index("c")
    sid = lax.axis_index("s")
    neg_inf = jnp.full((L,), -jnp.inf, dtype=jnp.float32)

    def scan_chunk(buf, st):
        def h_body(h, st):
            def wt_body(wtx, st):
                ms = [list(st[K * g:K * (g + 1)]) for g in range(NG)]
                for wsx in range(WS):
                    for g in range(NG):
                        v = buf[h, wtx, wsx, pl.ds(g * L, L)]
                        _insert(ms[g], v)
                return tuple(x for m in ms for x in m)

            return lax.fori_loop(0, WT, wt_body, st)

        return lax.fori_loop(0, CH, h_body, st)

    def task_body(k, _):
        u = (core * NS + sid) * TASKS_PER_TILE + k
        pair = u // 4
        q = u % 4
        b = pair // CT
        ct = pair % CT
        h0 = q * HQ

        pltpu.async_copy(
            x_hbm.at[b, pl.ds(h0, CH), :, ct], buf0, sem0)
        pltpu.async_copy(
            x_hbm.at[b, pl.ds(h0 + CH, CH), :, ct], buf1, sem1)

        def pair_body(g, st):
            hc = h0 + 2 * g * CH
            pltpu.make_async_copy(
                x_hbm.at[b, pl.ds(0, CH), :, ct], buf0, sem0).wait()
            st = scan_chunk(buf0, st)

            @pl.when(g < NCHUNK // 2 - 1)
            def _next0():
                pltpu.async_copy(
                    x_hbm.at[b, pl.ds(hc + 2 * CH, CH), :, ct], buf0, sem0)

            pltpu.make_async_copy(
                x_hbm.at[b, pl.ds(0, CH), :, ct], buf1, sem1).wait()
            st = scan_chunk(buf1, st)

            @pl.when(g < NCHUNK // 2 - 1)
            def _next1():
                pltpu.async_copy(
                    x_hbm.at[b, pl.ds(hc + 3 * CH, CH), :, ct], buf1, sem1)

            return st

        st = lax.fori_loop(0, NCHUNK // 2, pair_body, (neg_inf,) * (K * NG))

        for g in range(NG):
            for k5 in range(K):
                cand_v[k5, pl.ds(g * L, L)] = st[K * g + k5]
        pltpu.sync_copy(cand_v, cands_hbm.at[u])
        return 0

    lax.fori_loop(0, TASKS_PER_TILE, task_body, 0)


def _merge_body(cands_hbm, out_hbm, merge_v, res_v):
    wid = lax.axis_index("c") * NS + lax.axis_index("s")

    @pl.when(wid < NPAIR)
    def _merge():
        b = wid // CT
        ct = wid % CT
        pltpu.sync_copy(cands_hbm.at[pl.ds(wid * 4, 4)], merge_v)
        for g in range(NG):
            m = [merge_v[0, k5, pl.ds(g * L, L)] for k5 in range(K)]
            for q in range(1, 4):
                for k5 in range(K):
                    _insert(m, merge_v[q, k5, pl.ds(g * L, L)])
            res_v[pl.ds(g * L, L)] = (m[0] + m[1] + m[2] + m[3] + m[4]) / 5.0
        pltpu.sync_copy(res_v, out_hbm.at[pl.ds(b * C + ct * CL, CL)])


def kernel(x):
    xp = x.reshape(B, CT, CL, H, WT, WS).transpose(0, 3, 4, 1, 5, 2)
    mesh = plsc.VectorSubcoreMesh(core_axis_name="c", subcore_axis_name="s")
    cands = pl.kernel(
        _scan_body,
        out_type=jax.ShapeDtypeStruct((4 * NPAIR, K, CL), jnp.float32),
        mesh=mesh,
        scratch_types=[
            pltpu.VMEM((CH, WT, WS, CL), jnp.float32),
            pltpu.VMEM((CH, WT, WS, CL), jnp.float32),
            pltpu.VMEM((K, CL), jnp.float32),
            pltpu.SemaphoreType.DMA,
            pltpu.SemaphoreType.DMA,
        ],
    )(xp)
    out = pl.kernel(
        _merge_body,
        out_type=jax.ShapeDtypeStruct((B * C,), jnp.float32),
        mesh=mesh,
        scratch_types=[
            pltpu.VMEM((4, K, CL), jnp.float32),
            pltpu.VMEM((CL,), jnp.float32),
        ],
    )(cands)
    return out.reshape(B, C)
